# Initial kernel scaffold; baseline (speedup 1.0000x reference)
import jax
import jax.numpy as jnp
from jax.experimental import pallas as pl


def kernel(x, edge_index, W0, b0, W1, b1, W2, b2, Wo, bo):
    raise NotImplementedError("write your pallas kernel here")


# speedup vs baseline: 10.6367x; 10.6367x over previous
import jax
import jax.numpy as jnp
from jax import lax
from jax.experimental import pallas as pl
from jax.experimental.pallas import tpu as pltpu
from jax.experimental.pallas import tpu_sc as plsc

N_NODES = 10000
N_PAD = 10240
E = 320000
D = 128
N_CLASS = 40

NC = 2
NS = 16
EPT = E // (NC * NS)
CHUNK = 80
NCHUNK = EPT // CHUNK
SL = N_PAD // NS

ROW_BLK = 2560
TC_GRID = N_PAD // ROW_BLK


DEG_W = 128


def _deg_body(dst_hbm, zeros_hbm, ones_hbm, deg_out, dstv, onesbuf, degacc):
    c = lax.axis_index("c")
    s = lax.axis_index("s")
    g = c * NS + s

    pltpu.sync_copy(ones_hbm, onesbuf)
    pltpu.sync_copy(zeros_hbm.at[pl.ds(s * SL, SL)],
                    degacc.at[pl.ds(s * SL, SL)])
    plsc.subcore_barrier()

    def chunk_body(j, _):
        base = g * EPT + j * CHUNK
        pltpu.sync_copy(dst_hbm.at[pl.ds(base, CHUNK)], dstv)
        pltpu.sync_copy(onesbuf, degacc.at[dstv], add=True)
        return 0
    lax.fori_loop(0, NCHUNK, chunk_body, 0)

    plsc.subcore_barrier()
    pltpu.sync_copy(degacc.at[pl.ds(s * SL, SL)],
                    deg_out.at[pl.ds(c * N_PAD + s * SL, SL)])


def _deg_kernel(dst, zeros16w, ones_rows):
    mesh = plsc.VectorSubcoreMesh(core_axis_name="c", subcore_axis_name="s")
    return pl.kernel(
        _deg_body,
        out_type=jax.ShapeDtypeStruct((NC * N_PAD, DEG_W), jnp.float32),
        mesh=mesh,
        scratch_types=[
            pltpu.VMEM((CHUNK,), jnp.int32),
            pltpu.VMEM((CHUNK, DEG_W), jnp.float32),
            pltpu.VMEM_SHARED((N_PAD, DEG_W), jnp.float32),
        ],
    )(dst, zeros16w, ones_rows)


def _agg_body(src_hbm, dst_hbm, mh_hbm, zeros_hbm, out_hbm,
              srcv, dstv, rows, acc, sem):
    c = lax.axis_index("c")
    s = lax.axis_index("s")
    g = c * NS + s

    pltpu.sync_copy(zeros_hbm.at[pl.ds(s * SL, SL)], acc.at[pl.ds(s * SL, SL)])
    plsc.subcore_barrier()

    def chunk_body(j, _):
        base = g * EPT + j * CHUNK
        pltpu.sync_copy(src_hbm.at[pl.ds(base, CHUNK)], srcv)
        pltpu.sync_copy(dst_hbm.at[pl.ds(base, CHUNK)], dstv)
        pltpu.async_copy(mh_hbm.at[srcv], rows, sem).wait()
        pltpu.sync_copy(rows, acc.at[dstv], add=True)
        return 0
    lax.fori_loop(0, NCHUNK, chunk_body, 0)

    plsc.subcore_barrier()
    pltpu.sync_copy(acc.at[pl.ds(s * SL, SL)],
                    out_hbm.at[pl.ds(c * N_PAD + s * SL, SL)])


def _agg_kernel(src, dst, mh, zeros_pad):
    mesh = plsc.VectorSubcoreMesh(core_axis_name="c", subcore_axis_name="s")
    return pl.kernel(
        _agg_body,
        out_type=jax.ShapeDtypeStruct((NC * N_PAD, D), jnp.float32),
        mesh=mesh,
        scratch_types=[
            pltpu.VMEM((CHUNK,), jnp.int32),
            pltpu.VMEM((CHUNK,), jnp.int32),
            pltpu.VMEM((CHUNK, D), jnp.float32),
            pltpu.VMEM_SHARED((N_PAD, D), jnp.float32),
            pltpu.SemaphoreType.DMA,
        ],
    )(src, dst, mh, zeros_pad)


def _pre_body(x_ref, w_ref, degp_ref, m_ref, mh_ref, dinv_ref):
    deg = degp_ref[0, :, 0] + degp_ref[1, :, 0] + 1.0
    dinv = lax.rsqrt(deg)[:, None]
    m = jnp.dot(x_ref[...], w_ref[...], preferred_element_type=jnp.float32)
    m_ref[...] = m
    mh_ref[...] = m * dinv
    dinv_ref[...] = dinv


def _tc_pre(x_pad, W0, degp):
    return pl.pallas_call(
        _pre_body,
        grid=(TC_GRID,),
        in_specs=[
            pl.BlockSpec((ROW_BLK, D), lambda i: (i, 0)),
            pl.BlockSpec((D, D), lambda i: (0, 0)),
            pl.BlockSpec((NC, ROW_BLK, DEG_W), lambda i: (0, i, 0)),
        ],
        out_specs=[
            pl.BlockSpec((ROW_BLK, D), lambda i: (i, 0)),
            pl.BlockSpec((ROW_BLK, D), lambda i: (i, 0)),
            pl.BlockSpec((ROW_BLK, 1), lambda i: (i, 0)),
        ],
        out_shape=[
            jax.ShapeDtypeStruct((N_PAD, D), jnp.float32),
            jax.ShapeDtypeStruct((N_PAD, D), jnp.float32),
            jax.ShapeDtypeStruct((N_PAD, 1), jnp.float32),
        ],
    )(x_pad, W0, degp)


def _mid_body(p_ref, m_prev_ref, dinv_ref, b_ref, jk_ref, w_ref,
              m_ref, mh_ref, jk_out_ref):
    dinv = dinv_ref[...]
    agg = p_ref[0] + p_ref[1]
    xl = jnp.maximum(dinv * agg + dinv * dinv * m_prev_ref[...]
                     + b_ref[...][None, :], 0.0)
    jk_out_ref[...] = jnp.maximum(jk_ref[...], xl)
    m = jnp.dot(xl, w_ref[...], preferred_element_type=jnp.float32)
    m_ref[...] = m
    mh_ref[...] = m * dinv


def _tc_mid(p, m_prev, dinv, b, jk, W):
    return pl.pallas_call(
        _mid_body,
        grid=(TC_GRID,),
        in_specs=[
            pl.BlockSpec((NC, ROW_BLK, D), lambda i: (0, i, 0)),
            pl.BlockSpec((ROW_BLK, D), lambda i: (i, 0)),
            pl.BlockSpec((ROW_BLK, 1), lambda i: (i, 0)),
            pl.BlockSpec((D,), lambda i: (0,)),
            pl.BlockSpec((ROW_BLK, D), lambda i: (i, 0)),
            pl.BlockSpec((D, D), lambda i: (0, 0)),
        ],
        out_specs=[
            pl.BlockSpec((ROW_BLK, D), lambda i: (i, 0)),
            pl.BlockSpec((ROW_BLK, D), lambda i: (i, 0)),
            pl.BlockSpec((ROW_BLK, D), lambda i: (i, 0)),
        ],
        out_shape=[
            jax.ShapeDtypeStruct((N_PAD, D), jnp.float32),
            jax.ShapeDtypeStruct((N_PAD, D), jnp.float32),
            jax.ShapeDtypeStruct((N_PAD, D), jnp.float32),
        ],
    )(p, m_prev, dinv, b, jk, W)


def _post_body(p_ref, m_prev_ref, dinv_ref, b_ref, jk_ref, wo_ref, bo_ref,
               out_ref):
    dinv = dinv_ref[...]
    agg = p_ref[0] + p_ref[1]
    xl = jnp.maximum(dinv * agg + dinv * dinv * m_prev_ref[...]
                     + b_ref[...][None, :], 0.0)
    jk = jnp.maximum(jk_ref[...], xl)
    out_ref[...] = (jnp.dot(jk, wo_ref[...], preferred_element_type=jnp.float32)
                    + bo_ref[...][None, :])


def _tc_post(p, m_prev, dinv, b, jk, Wo, bo):
    return pl.pallas_call(
        _post_body,
        grid=(TC_GRID,),
        in_specs=[
            pl.BlockSpec((NC, ROW_BLK, D), lambda i: (0, i, 0)),
            pl.BlockSpec((ROW_BLK, D), lambda i: (i, 0)),
            pl.BlockSpec((ROW_BLK, 1), lambda i: (i, 0)),
            pl.BlockSpec((D,), lambda i: (0,)),
            pl.BlockSpec((ROW_BLK, D), lambda i: (i, 0)),
            pl.BlockSpec((D, N_CLASS), lambda i: (0, 0)),
            pl.BlockSpec((N_CLASS,), lambda i: (0,)),
        ],
        out_specs=[pl.BlockSpec((ROW_BLK, N_CLASS), lambda i: (i, 0))],
        out_shape=[jax.ShapeDtypeStruct((N_PAD, N_CLASS), jnp.float32)],
    )(p, m_prev, dinv, b, jk, Wo, bo)


def kernel(x, edge_index, W0, b0, W1, b1, W2, b2, Wo, bo):
    src = edge_index[0]
    dst = edge_index[1]
    x_pad = jnp.pad(x, ((0, N_PAD - N_NODES), (0, 0)))
    zeros_pad = jnp.zeros((N_PAD, D), jnp.float32)
    ones_rows = jnp.ones((CHUNK, DEG_W), jnp.float32)
    jk0 = jnp.zeros((N_PAD, D), jnp.float32)

    degp = _deg_kernel(dst, zeros_pad, ones_rows).reshape(NC, N_PAD, DEG_W)
    m0, mh0, dinv = _tc_pre(x_pad, W0, degp)

    p1 = _agg_kernel(src, dst, mh0, zeros_pad).reshape(NC, N_PAD, D)
    m1, mh1, jk1 = _tc_mid(p1, m0, dinv, b0, jk0, W1)

    p2 = _agg_kernel(src, dst, mh1, zeros_pad).reshape(NC, N_PAD, D)
    m2, mh2, jk2 = _tc_mid(p2, m1, dinv, b1, jk1, W2)

    p3 = _agg_kernel(src, dst, mh2, zeros_pad).reshape(NC, N_PAD, D)
    out = _tc_post(p3, m2, dinv, b2, jk2, Wo, bo)[0]

    return out[:N_NODES]


# speedup vs baseline: 23.1221x; 2.1738x over previous
import jax
import jax.numpy as jnp
from jax import lax
from jax.experimental import pallas as pl
from jax.experimental.pallas import tpu as pltpu
from jax.experimental.pallas import tpu_sc as plsc

N_NODES = 10000
N_PAD = 10240
E = 320000
D = 128
N_CLASS = 40

NC = 2
NS = 16
EPT = E // (NC * NS)
CHUNK = 80
NCHUNK = EPT // CHUNK
SL = N_PAD // NS

ROW_BLK = 2560
TC_GRID = N_PAD // ROW_BLK


DEG_W = 128


def _deg_body(dst2_hbm, zeros_hbm, ones_hbm, deg_out,
              dstbuf, onesbuf, degacc, sem):
    c = lax.axis_index("c")
    s = lax.axis_index("s")
    g = c * NS + s

    pltpu.sync_copy(ones_hbm, onesbuf)
    pltpu.sync_copy(dst2_hbm.at[g], dstbuf)
    pltpu.sync_copy(zeros_hbm.at[pl.ds(s * SL, SL)],
                    degacc.at[pl.ds(s * SL, SL)])
    plsc.subcore_barrier()

    def fire_body(j, _):
        pltpu.async_copy(onesbuf, degacc.at[dstbuf.at[j]], sem, add=True)
        return 0
    lax.fori_loop(0, NCHUNK, fire_body, 0)

    def drain_body(j, _):
        pltpu.make_async_copy(onesbuf, degacc.at[dstbuf.at[j]], sem).wait()
        return 0
    lax.fori_loop(0, NCHUNK, drain_body, 0)

    plsc.subcore_barrier()
    pltpu.sync_copy(degacc.at[pl.ds(s * SL, SL)],
                    deg_out.at[pl.ds(c * N_PAD + s * SL, SL)])


def _deg_kernel(dst2, zeros_pad, ones_rows):
    mesh = plsc.VectorSubcoreMesh(core_axis_name="c", subcore_axis_name="s")
    return pl.kernel(
        _deg_body,
        out_type=jax.ShapeDtypeStruct((NC * N_PAD, DEG_W), jnp.float32),
        mesh=mesh,
        scratch_types=[
            pltpu.VMEM((NCHUNK, CHUNK), jnp.int32),
            pltpu.VMEM((CHUNK, DEG_W), jnp.float32),
            pltpu.VMEM_SHARED((N_PAD, DEG_W), jnp.float32),
            pltpu.SemaphoreType.DMA,
        ],
    )(dst2, zeros_pad, ones_rows)


def _agg_body(src1_hbm, dst2_hbm, mh_hbm, zeros_hbm, out_hbm,
              srcbuf, dstbuf, rows_a, rows_b, acc, sem_a, sem_b):
    c = lax.axis_index("c")
    s = lax.axis_index("s")
    g = c * NS + s

    pltpu.sync_copy(src1_hbm.at[pl.ds(g * EPT, EPT)], srcbuf)
    pltpu.sync_copy(dst2_hbm.at[g], dstbuf)
    pltpu.sync_copy(zeros_hbm.at[pl.ds(s * SL, SL)], acc.at[pl.ds(s * SL, SL)])
    plsc.subcore_barrier()

    def fire(k, rbuf, sem):
        pltpu.async_copy(mh_hbm.at[srcbuf.at[pl.ds(k * CHUNK, CHUNK)]],
                         rbuf, sem)

    def wait(rbuf, sem):
        pltpu.make_async_copy(mh_hbm.at[srcbuf.at[pl.ds(0, CHUNK)]],
                              rbuf, sem).wait()

    def scatter(k, rbuf):
        pltpu.sync_copy(rbuf, acc.at[dstbuf.at[k]], add=True)

    fire(0, rows_a, sem_a)

    def pair_body(i, _):
        k = 2 * i
        fire(k + 1, rows_b, sem_b)
        wait(rows_a, sem_a)
        scatter(k, rows_a)
        fire(k + 2, rows_a, sem_a)
        wait(rows_b, sem_b)
        scatter(k + 1, rows_b)
        return 0
    lax.fori_loop(0, (NCHUNK - 1) // 2, pair_body, 0)

    wait(rows_a, sem_a)
    scatter(NCHUNK - 1, rows_a)

    plsc.subcore_barrier()
    pltpu.sync_copy(acc.at[pl.ds(s * SL, SL)],
                    out_hbm.at[pl.ds(c * N_PAD + s * SL, SL)])


def _agg_kernel(src1, dst2, mh, zeros_pad):
    mesh = plsc.VectorSubcoreMesh(core_axis_name="c", subcore_axis_name="s")
    return pl.kernel(
        _agg_body,
        out_type=jax.ShapeDtypeStruct((NC * N_PAD, D), jnp.float32),
        mesh=mesh,
        scratch_types=[
            pltpu.VMEM((EPT,), jnp.int32),
            pltpu.VMEM((NCHUNK, CHUNK), jnp.int32),
            pltpu.VMEM((CHUNK, D), jnp.float32),
            pltpu.VMEM((CHUNK, D), jnp.float32),
            pltpu.VMEM_SHARED((N_PAD, D), jnp.float32),
            pltpu.SemaphoreType.DMA,
            pltpu.SemaphoreType.DMA,
        ],
    )(src1, dst2, mh, zeros_pad)


def _pre_body(x_ref, w_ref, degp_ref, m_ref, mh_ref, dinv_ref):
    deg = degp_ref[0, :, 0] + degp_ref[1, :, 0] + 1.0
    dinv = lax.rsqrt(deg)[:, None]
    m = jnp.dot(x_ref[...], w_ref[...], preferred_element_type=jnp.float32)
    m_ref[...] = m
    mh_ref[...] = m * dinv
    dinv_ref[...] = dinv


def _tc_pre(x_pad, W0, degp):
    return pl.pallas_call(
        _pre_body,
        grid=(TC_GRID,),
        in_specs=[
            pl.BlockSpec((ROW_BLK, D), lambda i: (i, 0)),
            pl.BlockSpec((D, D), lambda i: (0, 0)),
            pl.BlockSpec((NC, ROW_BLK, DEG_W), lambda i: (0, i, 0)),
        ],
        out_specs=[
            pl.BlockSpec((ROW_BLK, D), lambda i: (i, 0)),
            pl.BlockSpec((ROW_BLK, D), lambda i: (i, 0)),
            pl.BlockSpec((ROW_BLK, 1), lambda i: (i, 0)),
        ],
        out_shape=[
            jax.ShapeDtypeStruct((N_PAD, D), jnp.float32),
            jax.ShapeDtypeStruct((N_PAD, D), jnp.float32),
            jax.ShapeDtypeStruct((N_PAD, 1), jnp.float32),
        ],
    )(x_pad, W0, degp)


def _mid_body(p_ref, m_prev_ref, dinv_ref, b_ref, jk_ref, w_ref,
              m_ref, mh_ref, jk_out_ref):
    dinv = dinv_ref[...]
    agg = p_ref[0] + p_ref[1]
    xl = jnp.maximum(dinv * agg + dinv * dinv * m_prev_ref[...]
                     + b_ref[...][None, :], 0.0)
    jk_out_ref[...] = jnp.maximum(jk_ref[...], xl)
    m = jnp.dot(xl, w_ref[...], preferred_element_type=jnp.float32)
    m_ref[...] = m
    mh_ref[...] = m * dinv


def _tc_mid(p, m_prev, dinv, b, jk, W):
    return pl.pallas_call(
        _mid_body,
        grid=(TC_GRID,),
        in_specs=[
            pl.BlockSpec((NC, ROW_BLK, D), lambda i: (0, i, 0)),
            pl.BlockSpec((ROW_BLK, D), lambda i: (i, 0)),
            pl.BlockSpec((ROW_BLK, 1), lambda i: (i, 0)),
            pl.BlockSpec((D,), lambda i: (0,)),
            pl.BlockSpec((ROW_BLK, D), lambda i: (i, 0)),
            pl.BlockSpec((D, D), lambda i: (0, 0)),
        ],
        out_specs=[
            pl.BlockSpec((ROW_BLK, D), lambda i: (i, 0)),
            pl.BlockSpec((ROW_BLK, D), lambda i: (i, 0)),
            pl.BlockSpec((ROW_BLK, D), lambda i: (i, 0)),
        ],
        out_shape=[
            jax.ShapeDtypeStruct((N_PAD, D), jnp.float32),
            jax.ShapeDtypeStruct((N_PAD, D), jnp.float32),
            jax.ShapeDtypeStruct((N_PAD, D), jnp.float32),
        ],
    )(p, m_prev, dinv, b, jk, W)


def _post_body(p_ref, m_prev_ref, dinv_ref, b_ref, jk_ref, wo_ref, bo_ref,
               out_ref):
    dinv = dinv_ref[...]
    agg = p_ref[0] + p_ref[1]
    xl = jnp.maximum(dinv * agg + dinv * dinv * m_prev_ref[...]
                     + b_ref[...][None, :], 0.0)
    jk = jnp.maximum(jk_ref[...], xl)
    out_ref[...] = (jnp.dot(jk, wo_ref[...], preferred_element_type=jnp.float32)
                    + bo_ref[...][None, :])


def _tc_post(p, m_prev, dinv, b, jk, Wo, bo):
    return pl.pallas_call(
        _post_body,
        grid=(TC_GRID,),
        in_specs=[
            pl.BlockSpec((NC, ROW_BLK, D), lambda i: (0, i, 0)),
            pl.BlockSpec((ROW_BLK, D), lambda i: (i, 0)),
            pl.BlockSpec((ROW_BLK, 1), lambda i: (i, 0)),
            pl.BlockSpec((D,), lambda i: (0,)),
            pl.BlockSpec((ROW_BLK, D), lambda i: (i, 0)),
            pl.BlockSpec((D, N_CLASS), lambda i: (0, 0)),
            pl.BlockSpec((N_CLASS,), lambda i: (0,)),
        ],
        out_specs=[pl.BlockSpec((ROW_BLK, N_CLASS), lambda i: (i, 0))],
        out_shape=[jax.ShapeDtypeStruct((N_PAD, N_CLASS), jnp.float32)],
    )(p, m_prev, dinv, b, jk, Wo, bo)


def kernel(x, edge_index, W0, b0, W1, b1, W2, b2, Wo, bo):
    src1 = edge_index[0]
    dst2 = edge_index[1].reshape(NC * NS, NCHUNK, CHUNK)
    x_pad = jnp.pad(x, ((0, N_PAD - N_NODES), (0, 0)))
    zeros_pad = jnp.zeros((N_PAD, D), jnp.float32)
    ones_rows = jnp.ones((CHUNK, DEG_W), jnp.float32)
    jk0 = jnp.zeros((N_PAD, D), jnp.float32)

    degp = _deg_kernel(dst2, zeros_pad, ones_rows).reshape(NC, N_PAD, DEG_W)
    m0, mh0, dinv = _tc_pre(x_pad, W0, degp)

    p1 = _agg_kernel(src1, dst2, mh0, zeros_pad).reshape(NC, N_PAD, D)
    m1, mh1, jk1 = _tc_mid(p1, m0, dinv, b0, jk0, W1)

    p2 = _agg_kernel(src1, dst2, mh1, zeros_pad).reshape(NC, N_PAD, D)
    m2, mh2, jk2 = _tc_mid(p2, m1, dinv, b1, jk1, W2)

    p3 = _agg_kernel(src1, dst2, mh2, zeros_pad).reshape(NC, N_PAD, D)
    out = _tc_post(p3, m2, dinv, b2, jk2, Wo, bo)[0]

    return out[:N_NODES]


# speedup vs baseline: 23.2678x; 1.0063x over previous
import jax
import jax.numpy as jnp
from jax import lax
from jax.experimental import pallas as pl
from jax.experimental.pallas import tpu as pltpu
from jax.experimental.pallas import tpu_sc as plsc

N_NODES = 10000
N_PAD = 10240
E = 320000
D = 128
N_CLASS = 40

NC = 2
NS = 16
EPT = E // (NC * NS)
CHUNK = 80
NCHUNK = EPT // CHUNK
SL = N_PAD // NS

ROW_BLK = 2560
TC_GRID = N_PAD // ROW_BLK


DEG_W = 128


def _deg_body(dst2_hbm, zeros_hbm, ones_hbm, deg_out,
              dstbuf, onesbuf, degacc, sem):
    c = lax.axis_index("c")
    s = lax.axis_index("s")
    g = c * NS + s

    pltpu.sync_copy(ones_hbm, onesbuf)
    pltpu.sync_copy(dst2_hbm.at[g], dstbuf)
    pltpu.sync_copy(zeros_hbm.at[pl.ds(s * SL, SL)],
                    degacc.at[pl.ds(s * SL, SL)])
    plsc.subcore_barrier()

    def fire_body(j, _):
        pltpu.async_copy(onesbuf, degacc.at[dstbuf.at[j]], sem, add=True)
        return 0
    lax.fori_loop(0, NCHUNK, fire_body, 0)

    def drain_body(j, _):
        pltpu.make_async_copy(onesbuf, degacc.at[dstbuf.at[j]], sem).wait()
        return 0
    lax.fori_loop(0, NCHUNK, drain_body, 0)

    plsc.subcore_barrier()
    pltpu.sync_copy(degacc.at[pl.ds(s * SL, SL)],
                    deg_out.at[pl.ds(c * N_PAD + s * SL, SL)])


def _deg_kernel(dst2, zeros_pad, ones_rows):
    mesh = plsc.VectorSubcoreMesh(core_axis_name="c", subcore_axis_name="s")
    return pl.kernel(
        _deg_body,
        out_type=jax.ShapeDtypeStruct((NC * N_PAD, DEG_W), jnp.float32),
        mesh=mesh,
        scratch_types=[
            pltpu.VMEM((NCHUNK, CHUNK), jnp.int32),
            pltpu.VMEM((CHUNK, DEG_W), jnp.float32),
            pltpu.VMEM_SHARED((N_PAD, DEG_W), jnp.float32),
            pltpu.SemaphoreType.DMA,
        ],
    )(dst2, zeros_pad, ones_rows)


def _agg_body(src1_hbm, dst2_hbm, mh_hbm, zeros_hbm, out_hbm,
              srcbuf, dstbuf, rows_a, rows_b, acc, sem_a, sem_b):
    c = lax.axis_index("c")
    s = lax.axis_index("s")
    g = c * NS + s

    pltpu.sync_copy(src1_hbm.at[pl.ds(g * EPT, EPT)], srcbuf)
    pltpu.sync_copy(dst2_hbm.at[g], dstbuf)
    pltpu.sync_copy(zeros_hbm.at[pl.ds(s * SL, SL)], acc.at[pl.ds(s * SL, SL)])
    plsc.subcore_barrier()

    HALF = CHUNK // 2

    def fire(k, rbuf, sem):
        pltpu.async_copy(mh_hbm.at[srcbuf.at[pl.ds(k * CHUNK, HALF)]],
                         rbuf.at[pl.ds(0, HALF)], sem)
        pltpu.async_copy(mh_hbm.at[srcbuf.at[pl.ds(k * CHUNK + HALF, HALF)]],
                         rbuf.at[pl.ds(HALF, HALF)], sem)

    def wait(rbuf, sem):
        pltpu.make_async_copy(mh_hbm.at[srcbuf.at[pl.ds(0, HALF)]],
                              rbuf.at[pl.ds(0, HALF)], sem).wait()
        pltpu.make_async_copy(mh_hbm.at[srcbuf.at[pl.ds(0, HALF)]],
                              rbuf.at[pl.ds(HALF, HALF)], sem).wait()

    def scatter(k, rbuf):
        pltpu.sync_copy(rbuf, acc.at[dstbuf.at[k]], add=True)

    fire(0, rows_a, sem_a)

    def pair_body(i, _):
        k = 2 * i
        fire(k + 1, rows_b, sem_b)
        wait(rows_a, sem_a)
        scatter(k, rows_a)
        fire(k + 2, rows_a, sem_a)
        wait(rows_b, sem_b)
        scatter(k + 1, rows_b)
        return 0
    lax.fori_loop(0, (NCHUNK - 1) // 2, pair_body, 0)

    wait(rows_a, sem_a)
    scatter(NCHUNK - 1, rows_a)

    plsc.subcore_barrier()
    pltpu.sync_copy(acc.at[pl.ds(s * SL, SL)],
                    out_hbm.at[pl.ds(c * N_PAD + s * SL, SL)])


def _agg_kernel(src1, dst2, mh, zeros_pad):
    mesh = plsc.VectorSubcoreMesh(core_axis_name="c", subcore_axis_name="s")
    return pl.kernel(
        _agg_body,
        out_type=jax.ShapeDtypeStruct((NC * N_PAD, D), jnp.float32),
        mesh=mesh,
        scratch_types=[
            pltpu.VMEM((EPT,), jnp.int32),
            pltpu.VMEM((NCHUNK, CHUNK), jnp.int32),
            pltpu.VMEM((CHUNK, D), jnp.float32),
            pltpu.VMEM((CHUNK, D), jnp.float32),
            pltpu.VMEM_SHARED((N_PAD, D), jnp.float32),
            pltpu.SemaphoreType.DMA,
            pltpu.SemaphoreType.DMA,
        ],
    )(src1, dst2, mh, zeros_pad)


def _m0_body(x_ref, w_ref, m_ref):
    m_ref[...] = jnp.dot(x_ref[...], w_ref[...],
                         preferred_element_type=jnp.float32)


def _tc_m0(x_pad, W0):
    return pl.pallas_call(
        _m0_body,
        grid=(TC_GRID,),
        in_specs=[
            pl.BlockSpec((ROW_BLK, D), lambda i: (i, 0)),
            pl.BlockSpec((D, D), lambda i: (0, 0)),
        ],
        out_specs=[pl.BlockSpec((ROW_BLK, D), lambda i: (i, 0))],
        out_shape=[jax.ShapeDtypeStruct((N_PAD, D), jnp.float32)],
    )(x_pad, W0)


def _scale_body(m_ref, degp_ref, mh_ref, dinv_ref):
    deg = degp_ref[0, :, 0] + degp_ref[1, :, 0] + 1.0
    dinv = lax.rsqrt(deg)[:, None]
    mh_ref[...] = m_ref[...] * dinv
    dinv_ref[...] = dinv


def _tc_scale(m0, degp):
    return pl.pallas_call(
        _scale_body,
        grid=(TC_GRID,),
        in_specs=[
            pl.BlockSpec((ROW_BLK, D), lambda i: (i, 0)),
            pl.BlockSpec((NC, ROW_BLK, DEG_W), lambda i: (0, i, 0)),
        ],
        out_specs=[
            pl.BlockSpec((ROW_BLK, D), lambda i: (i, 0)),
            pl.BlockSpec((ROW_BLK, 1), lambda i: (i, 0)),
        ],
        out_shape=[
            jax.ShapeDtypeStruct((N_PAD, D), jnp.float32),
            jax.ShapeDtypeStruct((N_PAD, 1), jnp.float32),
        ],
    )(m0, degp)


def _mid_body(p_ref, m_prev_ref, dinv_ref, b_ref, jk_ref, w_ref,
              m_ref, mh_ref, jk_out_ref):
    dinv = dinv_ref[...]
    agg = p_ref[0] + p_ref[1]
    xl = jnp.maximum(dinv * agg + dinv * dinv * m_prev_ref[...]
                     + b_ref[...][None, :], 0.0)
    jk_out_ref[...] = jnp.maximum(jk_ref[...], xl)
    m = jnp.dot(xl, w_ref[...], preferred_element_type=jnp.float32)
    m_ref[...] = m
    mh_ref[...] = m * dinv


def _tc_mid(p, m_prev, dinv, b, jk, W):
    return pl.pallas_call(
        _mid_body,
        grid=(TC_GRID,),
        in_specs=[
            pl.BlockSpec((NC, ROW_BLK, D), lambda i: (0, i, 0)),
            pl.BlockSpec((ROW_BLK, D), lambda i: (i, 0)),
            pl.BlockSpec((ROW_BLK, 1), lambda i: (i, 0)),
            pl.BlockSpec((D,), lambda i: (0,)),
            pl.BlockSpec((ROW_BLK, D), lambda i: (i, 0)),
            pl.BlockSpec((D, D), lambda i: (0, 0)),
        ],
        out_specs=[
            pl.BlockSpec((ROW_BLK, D), lambda i: (i, 0)),
            pl.BlockSpec((ROW_BLK, D), lambda i: (i, 0)),
            pl.BlockSpec((ROW_BLK, D), lambda i: (i, 0)),
        ],
        out_shape=[
            jax.ShapeDtypeStruct((N_PAD, D), jnp.float32),
            jax.ShapeDtypeStruct((N_PAD, D), jnp.float32),
            jax.ShapeDtypeStruct((N_PAD, D), jnp.float32),
        ],
    )(p, m_prev, dinv, b, jk, W)


def _post_body(p_ref, m_prev_ref, dinv_ref, b_ref, jk_ref, wo_ref, bo_ref,
               out_ref):
    dinv = dinv_ref[...]
    agg = p_ref[0] + p_ref[1]
    xl = jnp.maximum(dinv * agg + dinv * dinv * m_prev_ref[...]
                     + b_ref[...][None, :], 0.0)
    jk = jnp.maximum(jk_ref[...], xl)
    out_ref[...] = (jnp.dot(jk, wo_ref[...], preferred_element_type=jnp.float32)
                    + bo_ref[...][None, :])


def _tc_post(p, m_prev, dinv, b, jk, Wo, bo):
    return pl.pallas_call(
        _post_body,
        grid=(TC_GRID,),
        in_specs=[
            pl.BlockSpec((NC, ROW_BLK, D), lambda i: (0, i, 0)),
            pl.BlockSpec((ROW_BLK, D), lambda i: (i, 0)),
            pl.BlockSpec((ROW_BLK, 1), lambda i: (i, 0)),
            pl.BlockSpec((D,), lambda i: (0,)),
            pl.BlockSpec((ROW_BLK, D), lambda i: (i, 0)),
            pl.BlockSpec((D, N_CLASS), lambda i: (0, 0)),
            pl.BlockSpec((N_CLASS,), lambda i: (0,)),
        ],
        out_specs=[pl.BlockSpec((ROW_BLK, N_CLASS), lambda i: (i, 0))],
        out_shape=[jax.ShapeDtypeStruct((N_PAD, N_CLASS), jnp.float32)],
    )(p, m_prev, dinv, b, jk, Wo, bo)


def kernel(x, edge_index, W0, b0, W1, b1, W2, b2, Wo, bo):
    src1 = edge_index[0]
    dst2 = edge_index[1].reshape(NC * NS, NCHUNK, CHUNK)
    x_pad = jnp.pad(x, ((0, N_PAD - N_NODES), (0, 0)))
    zeros_pad = jnp.zeros((N_PAD, D), jnp.float32)
    ones_rows = jnp.ones((CHUNK, DEG_W), jnp.float32)
    jk0 = jnp.zeros((N_PAD, D), jnp.float32)

    m0 = _tc_m0(x_pad, W0)[0]
    degp = _deg_kernel(dst2, zeros_pad, ones_rows).reshape(NC, N_PAD, DEG_W)
    mh0, dinv = _tc_scale(m0, degp)

    p1 = _agg_kernel(src1, dst2, mh0, zeros_pad).reshape(NC, N_PAD, D)
    m1, mh1, jk1 = _tc_mid(p1, m0, dinv, b0, jk0, W1)

    p2 = _agg_kernel(src1, dst2, mh1, zeros_pad).reshape(NC, N_PAD, D)
    m2, mh2, jk2 = _tc_mid(p2, m1, dinv, b1, jk1, W2)

    p3 = _agg_kernel(src1, dst2, mh2, zeros_pad).reshape(NC, N_PAD, D)
    out = _tc_post(p3, m2, dinv, b2, jk2, Wo, bo)[0]

    return out[:N_NODES]
